# baseline (speedup 1.0000x reference)
import jax
import jax.numpy as jnp
from jax import lax
from jax.experimental import pallas as pl
from jax.experimental.pallas import tpu as pltpu
from jax.experimental.pallas import tpu_sc as plsc

_NC = 2
_NS = 16
_NW = _NC * _NS
_ICH = 128
_L = 16
_BL = 12800

_EPS = 1e-5


def _bf16_hi_bits(x):
    r = lax.convert_element_type(lax.convert_element_type(x, jnp.bfloat16),
                                 jnp.float32)
    return lax.bitcast_convert_type(r, jnp.uint32)


def _repack_body(a_ref, b_ref, c_ref, d_ref, out_ref):
    a, b, c, d = a_ref[...], b_ref[...], c_ref[...], d_ref[...]
    lo_words = _bf16_hi_bits(a) | (_bf16_hi_bits(c) >> 16)
    hi_words = _bf16_hi_bits(b) | (_bf16_hi_bits(d) >> 16)
    w = jnp.concatenate([lo_words, hi_words], axis=0)
    out_ref[...] = lax.bitcast_convert_type(jnp.transpose(w, (1, 0)),
                                            jnp.float32)


def _seg_len(N):
    return _BL * (-(-(N // 4) // _BL))


def _repack(tbl_t):
    D, N = tbl_t.shape
    A = _seg_len(N)
    nb = A // _BL
    last = (N - 1) // _BL
    blk = lambda off: pl.BlockSpec(
        (D, _BL), lambda i: (0, jnp.minimum(i + off, last)))
    return pl.pallas_call(
        _repack_body,
        grid=(nb,),
        in_specs=[blk(0), blk(nb), blk(2 * nb), blk(3 * nb)],
        out_specs=pl.BlockSpec((_BL, 2 * D), lambda i: (i, 0)),
        out_shape=jax.ShapeDtypeStruct((A, 2 * D), jnp.float32),
    )(tbl_t, tbl_t, tbl_t, tbl_t)


def _sc_gather_body(idx_hbm, tbl_hbm, out_hbm, idx_v, quad_v, rows, sem):
    bpw = idx_v.shape[0]
    nch = bpw // _ICH
    wid = lax.axis_index("s") * _NC + lax.axis_index("c")
    base = wid * bpw
    A = tbl_hbm.shape[0]
    pltpu.sync_copy(idx_hbm.at[pl.ds(base, bpw)], idx_v)

    def mk_quads(g, carry):
        ig = idx_v[pl.ds(g * _L, _L)]
        p = jnp.where(ig >= 3 * A, ig - 3 * A,
                      jnp.where(ig >= 2 * A, ig - 2 * A,
                                jnp.where(ig >= A, ig - A, ig)))
        quad_v[pl.ds(g * _L, _L)] = p
        return carry

    lax.fori_loop(0, bpw // _L, mk_quads, 0, unroll=False)

    copies = [pltpu.async_copy(
        tbl_hbm.at[quad_v.at[pl.ds(ch * _ICH, _ICH)]],
        rows.at[pl.ds(ch * _ICH, _ICH)], sem) for ch in range(nch)]
    for cp in copies:
        cp.wait()
    pltpu.sync_copy(rows, out_hbm.at[pl.ds(base, bpw)])


def _sc_gather(idx, tbl_p, B):
    bpw = B // _NW
    mesh = plsc.VectorSubcoreMesh(core_axis_name="c", subcore_axis_name="s")
    f = pl.kernel(
        _sc_gather_body,
        out_type=jax.ShapeDtypeStruct((B, tbl_p.shape[1]), tbl_p.dtype),
        mesh=mesh,
        scratch_types=[
            pltpu.VMEM((bpw,), jnp.int32),
            pltpu.VMEM((bpw,), jnp.int32),
            pltpu.VMEM((bpw, tbl_p.shape[1]), tbl_p.dtype),
            pltpu.SemaphoreType.DMA,
        ],
        compiler_params=pltpu.CompilerParams(needs_layout_passes=False),
    )
    return f(idx, tbl_p)


def _ln(x):
    mu = jnp.mean(x, axis=-1, keepdims=True)
    var = jnp.mean((x - mu) * (x - mu), axis=-1, keepdims=True)
    return (x - mu) * lax.rsqrt(var + _EPS)


def _gelu(x):
    return x * 0.5 * (1.0 + lax.erf(x * 0.7071067811865476))


def _unpack(pair_ref, q_ref, D):
    q = q_ref[...].reshape(-1, 1)
    words = jnp.where((q & 1) == 1, pair_ref[:, D:], pair_ref[:, :D])
    u = lax.bitcast_convert_type(words, jnp.uint32)
    bits = jnp.where((q >> 1) == 1, u << 16, u & jnp.uint32(0xFFFF0000))
    return lax.bitcast_convert_type(bits, jnp.float32)


def _mlp_body(idp_ref, namep_ref, qid_ref, qname_ref, cty_ref, ecty_ref,
              w1a_ref, w1b_ref, w1c_ref, b1_ref, w2_ref, b2_ref,
              w3_ref, b3_ref, out_ref):
    f32 = jnp.float32
    D = w1a_ref.shape[0]
    id_emb = _unpack(idp_ref, qid_ref, D)
    name_emb = _unpack(namep_ref, qname_ref, D)
    cty = cty_ref[...]
    ncty = ecty_ref.shape[0]
    bf16 = jnp.bfloat16
    onehot = jnp.where(
        cty.reshape(-1, 1) == lax.broadcasted_iota(jnp.int32, (1, ncty), 1),
        f32(1.0), f32(0.0)).astype(bf16)
    cty_emb = jnp.dot(onehot, ecty_ref[...].astype(bf16),
                      preferred_element_type=f32)
    h = (jnp.dot(id_emb.astype(bf16), w1a_ref[...].astype(bf16),
                 preferred_element_type=f32)
         + jnp.dot(cty_emb.astype(bf16), w1b_ref[...].astype(bf16),
                   preferred_element_type=f32)
         + jnp.dot(name_emb.astype(bf16), w1c_ref[...].astype(bf16),
                   preferred_element_type=f32)
         + b1_ref[...])
    h = _gelu(_ln(h))
    h = jnp.dot(h, w2_ref[...], preferred_element_type=f32) + b2_ref[...]
    h = _gelu(_ln(h))
    h = jnp.dot(h, w3_ref[...], preferred_element_type=f32) + b3_ref[...]
    out_ref[...] = _gelu(h)


def _mlp(id_pair, name_pair, qid3, qname3, cty3, E_cty,
         W1, b1, W2, b2, W3, b3, block_b):
    B = id_pair.shape[0]
    D = E_cty.shape[1]
    NCTY = E_cty.shape[0]
    H1 = W1.shape[1]
    H2 = W2.shape[1]
    H3 = W3.shape[1]
    grid = (B // block_b,)
    pair = lambda: pl.BlockSpec((block_b, 2 * D), lambda i: (i, 0))
    ivec = lambda: pl.BlockSpec((block_b,), lambda i: (i,))
    full = lambda r, c: pl.BlockSpec((r, c), lambda i: (0, 0))
    return pl.pallas_call(
        _mlp_body,
        grid=grid,
        in_specs=[
            pair(), pair(), ivec(), ivec(), ivec(),
            full(NCTY, D),
            full(D, H1), full(D, H1), full(D, H1), full(1, H1),
            full(H1, H2), full(1, H2),
            full(H2, H3), full(1, H3),
        ],
        out_specs=pl.BlockSpec((block_b, H3), lambda i: (i, 0)),
        out_shape=jax.ShapeDtypeStruct((B, H3), jnp.float32),
    )(id_pair, name_pair, qid3, qname3, cty3, E_cty,
      W1[:D], W1[D:2 * D], W1[2 * D:], b1.reshape(1, H1),
      W2, b2.reshape(1, H2), W3, b3.reshape(1, H3))


def kernel(user_ids, user_countries, user_names, E_id, E_cty, E_name,
           W1, b1, W2, b2, W3, b3):
    B = user_ids.shape[0]
    block_b = 4096
    nblk = B // block_b
    ids = user_ids.astype(jnp.int32)
    names = user_names.astype(jnp.int32)
    A_id = _seg_len(E_id.shape[0])
    A_name = _seg_len(E_name.shape[0])
    ename_p = _repack(E_name.T)
    name_pair = _sc_gather(names, ename_p, B)
    eid_p = _repack(E_id.T)
    id_pair = _sc_gather(ids, eid_p, B)

    def seg_code(idx, A):
        return ((idx >= A).astype(jnp.int32) + (idx >= 2 * A).astype(jnp.int32)
                + (idx >= 3 * A).astype(jnp.int32))

    qid3 = seg_code(ids, A_id)
    qname3 = seg_code(names, A_name)
    cty3 = user_countries.astype(jnp.int32)
    return _mlp(id_pair, name_pair, qid3, qname3, cty3, E_cty,
                W1, b1, W2, b2, W3, b3, block_b)


# speedup vs baseline: 1.0170x; 1.0170x over previous
import jax
import jax.numpy as jnp
from jax import lax
from jax.experimental import pallas as pl
from jax.experimental.pallas import tpu as pltpu
from jax.experimental.pallas import tpu_sc as plsc

_NC = 2
_NS = 16
_NW = _NC * _NS
_ICH = 128
_L = 16
_BL = 12800

_EPS = 1e-5


def _bf16_hi_bits(x):
    r = lax.convert_element_type(lax.convert_element_type(x, jnp.bfloat16),
                                 jnp.float32)
    return lax.bitcast_convert_type(r, jnp.uint32)


def _repack_body(a_ref, b_ref, c_ref, d_ref, out_ref):
    a, b, c, d = a_ref[...], b_ref[...], c_ref[...], d_ref[...]
    lo_words = _bf16_hi_bits(a) | (_bf16_hi_bits(c) >> 16)
    hi_words = _bf16_hi_bits(b) | (_bf16_hi_bits(d) >> 16)
    w = jnp.concatenate([lo_words, hi_words], axis=0)
    out_ref[...] = lax.bitcast_convert_type(jnp.transpose(w, (1, 0)),
                                            jnp.float32)


def _seg_len(N):
    return _BL * (-(-(N // 4) // _BL))


def _repack(tbl_t):
    D, N = tbl_t.shape
    A = _seg_len(N)
    nb = A // _BL
    last = (N - 1) // _BL
    blk = lambda off: pl.BlockSpec(
        (D, _BL), lambda i: (0, jnp.minimum(i + off, last)))
    return pl.pallas_call(
        _repack_body,
        grid=(nb,),
        in_specs=[blk(0), blk(nb), blk(2 * nb), blk(3 * nb)],
        out_specs=pl.BlockSpec((_BL, 2 * D), lambda i: (i, 0)),
        out_shape=jax.ShapeDtypeStruct((A, 2 * D), jnp.float32),
    )(tbl_t, tbl_t, tbl_t, tbl_t)


def _sc_gather_body(idx_hbm, tbl_hbm, out_hbm, idx_v, quad_v, rows, sem):
    bpw = idx_v.shape[0]
    nch = bpw // _ICH
    wid = lax.axis_index("s") * _NC + lax.axis_index("c")
    base = wid * bpw
    A = tbl_hbm.shape[0]
    pltpu.sync_copy(idx_hbm.at[pl.ds(base, bpw)], idx_v)

    def mk_quads(g, carry):
        ig = idx_v[pl.ds(g * _L, _L)]
        p = jnp.where(ig >= 3 * A, ig - 3 * A,
                      jnp.where(ig >= 2 * A, ig - 2 * A,
                                jnp.where(ig >= A, ig - A, ig)))
        quad_v[pl.ds(g * _L, _L)] = p
        return carry

    lax.fori_loop(0, bpw // _L, mk_quads, 0, unroll=False)

    copies = [pltpu.async_copy(
        tbl_hbm.at[quad_v.at[pl.ds(ch * _ICH, _ICH)]],
        rows.at[pl.ds(ch * _ICH, _ICH)], sem) for ch in range(nch)]
    for cp in copies:
        cp.wait()
    pltpu.sync_copy(rows, out_hbm.at[pl.ds(base, bpw)])


def _sc_gather(idx, tbl_p, B):
    bpw = B // _NW
    mesh = plsc.VectorSubcoreMesh(core_axis_name="c", subcore_axis_name="s")
    f = pl.kernel(
        _sc_gather_body,
        out_type=jax.ShapeDtypeStruct((B, tbl_p.shape[1]), tbl_p.dtype),
        mesh=mesh,
        scratch_types=[
            pltpu.VMEM((bpw,), jnp.int32),
            pltpu.VMEM((bpw,), jnp.int32),
            pltpu.VMEM((bpw, tbl_p.shape[1]), tbl_p.dtype),
            pltpu.SemaphoreType.DMA,
        ],
        compiler_params=pltpu.CompilerParams(needs_layout_passes=False),
    )
    return f(idx, tbl_p)


def _ln(x):
    mu = jnp.mean(x, axis=-1, keepdims=True)
    var = jnp.mean((x - mu) * (x - mu), axis=-1, keepdims=True)
    return (x - mu) * lax.rsqrt(var + _EPS)


def _gelu(x):
    return x * 0.5 * (1.0 + lax.erf(x * 0.7071067811865476))


def _unpack(pair_ref, q_ref, D):
    q = q_ref[...].reshape(-1, 1)
    words = jnp.where((q & 1) == 1, pair_ref[:, D:], pair_ref[:, :D])
    u = lax.bitcast_convert_type(words, jnp.uint32)
    bits = jnp.where((q >> 1) == 1, u << 16, u & jnp.uint32(0xFFFF0000))
    return lax.bitcast_convert_type(bits, jnp.float32)


def _mlp_body(idp_ref, namep_ref, qid_ref, qname_ref, cty_ref, ecty_ref,
              w1a_ref, w1b_ref, w1c_ref, b1_ref, w2_ref, b2_ref,
              w3_ref, b3_ref, out_ref):
    f32 = jnp.float32
    D = w1a_ref.shape[0]
    id_emb = _unpack(idp_ref, qid_ref, D)
    name_emb = _unpack(namep_ref, qname_ref, D)
    cty = cty_ref[...]
    ncty = ecty_ref.shape[0]
    bf16 = jnp.bfloat16
    onehot = jnp.where(
        cty.reshape(-1, 1) == lax.broadcasted_iota(jnp.int32, (1, ncty), 1),
        f32(1.0), f32(0.0)).astype(bf16)
    cty_emb = jnp.dot(onehot, ecty_ref[...].astype(bf16),
                      preferred_element_type=f32)
    h = (jnp.dot(id_emb, w1a_ref[...], preferred_element_type=f32)
         + jnp.dot(cty_emb, w1b_ref[...], preferred_element_type=f32)
         + jnp.dot(name_emb, w1c_ref[...], preferred_element_type=f32)
         + b1_ref[...])
    h = _gelu(_ln(h))
    h = jnp.dot(h, w2_ref[...], preferred_element_type=f32) + b2_ref[...]
    h = _gelu(_ln(h))
    h = jnp.dot(h, w3_ref[...], preferred_element_type=f32) + b3_ref[...]
    out_ref[...] = _gelu(h)


def _mlp(id_pair, name_pair, qid3, qname3, cty3, E_cty,
         W1, b1, W2, b2, W3, b3, block_b):
    B = id_pair.shape[0]
    D = E_cty.shape[1]
    NCTY = E_cty.shape[0]
    H1 = W1.shape[1]
    H2 = W2.shape[1]
    H3 = W3.shape[1]
    grid = (B // block_b,)
    pair = lambda: pl.BlockSpec((block_b, 2 * D), lambda i: (i, 0))
    ivec = lambda: pl.BlockSpec((block_b,), lambda i: (i,))
    full = lambda r, c: pl.BlockSpec((r, c), lambda i: (0, 0))
    return pl.pallas_call(
        _mlp_body,
        grid=grid,
        in_specs=[
            pair(), pair(), ivec(), ivec(), ivec(),
            full(NCTY, D),
            full(D, H1), full(D, H1), full(D, H1), full(1, H1),
            full(H1, H2), full(1, H2),
            full(H2, H3), full(1, H3),
        ],
        out_specs=pl.BlockSpec((block_b, H3), lambda i: (i, 0)),
        out_shape=jax.ShapeDtypeStruct((B, H3), jnp.float32),
    )(id_pair, name_pair, qid3, qname3, cty3, E_cty,
      W1[:D], W1[D:2 * D], W1[2 * D:], b1.reshape(1, H1),
      W2, b2.reshape(1, H2), W3, b3.reshape(1, H3))


def kernel(user_ids, user_countries, user_names, E_id, E_cty, E_name,
           W1, b1, W2, b2, W3, b3):
    B = user_ids.shape[0]
    block_b = 4096
    nblk = B // block_b
    ids = user_ids.astype(jnp.int32)
    names = user_names.astype(jnp.int32)
    A_id = _seg_len(E_id.shape[0])
    A_name = _seg_len(E_name.shape[0])
    ename_p = _repack(E_name.T)
    name_pair = _sc_gather(names, ename_p, B)
    eid_p = _repack(E_id.T)
    id_pair = _sc_gather(ids, eid_p, B)

    def seg_code(idx, A):
        return ((idx >= A).astype(jnp.int32) + (idx >= 2 * A).astype(jnp.int32)
                + (idx >= 3 * A).astype(jnp.int32))

    qid3 = seg_code(ids, A_id)
    qname3 = seg_code(names, A_name)
    cty3 = user_countries.astype(jnp.int32)
    return _mlp(id_pair, name_pair, qid3, qname3, cty3, E_cty,
                W1, b1, W2, b2, W3, b3, block_b)
